# Initial kernel scaffold; baseline (speedup 1.0000x reference)
import jax
import jax.numpy as jnp
from jax.experimental import pallas as pl


def kernel(x, edge_index, imu_data, params):
    raise NotImplementedError("write your pallas kernel here")


# speedup vs baseline: 49.0144x; 49.0144x over previous
import jax
import jax.numpy as jnp
from jax import lax
from jax.experimental import pallas as pl

_N = 512
_F0 = 8
_T = 8
_B = 4
_E = 8192
_C = 64
_K = 3
_f32 = jnp.float32
_HI = lax.Precision.HIGHEST
_DEF = lax.Precision.DEFAULT


def _mm(a, b, prec=_HI):
    return lax.dot_general(a, b, (((a.ndim - 1,), (0,)), ((), ())),
                           precision=prec, preferred_element_type=_f32)


def _mm00(a, b, prec=_HI):
    return lax.dot_general(a, b, (((0,), (0,)), ((), ())),
                           precision=prec, preferred_element_type=_f32)


def _mm11(a, b, prec=_HI):
    return lax.dot_general(a, b, (((1,), (1,)), ((), ())),
                           precision=prec, preferred_element_type=_f32)


def _iota_r(n):
    return lax.broadcasted_iota(jnp.int32, (n, 1), 0).astype(_f32)


def _iota_c(n):
    return lax.broadcasted_iota(jnp.int32, (1, n), 1).astype(_f32)


def _pool_body(x_ref, w_ref, out_ref, rank_ref):
    xb = x_ref[0]
    w = w_ref[...]
    nrm = jnp.sqrt(jnp.sum(w * w))
    s = jnp.tanh(jnp.sum(xb * w, axis=1, keepdims=True) / nrm)
    eye = (_iota_r(_N) == _iota_c(_N)).astype(_f32)
    s_row = _mm00(s, eye)
    gt = (s > s_row).astype(_f32)
    tie = ((s == s_row) & (_iota_r(_N) < _iota_c(_N))).astype(_f32)
    rank = jnp.sum(gt + tie, axis=0, keepdims=True)
    perm = (rank == _iota_r(_N)).astype(_f32)
    out_ref[0] = _mm(perm, xb * s)
    rank_ref[0] = rank


def _pool_call(x_tn, w):
    return pl.pallas_call(
        _pool_body,
        grid=(_B * _T,),
        in_specs=[
            pl.BlockSpec((1, _N, _F0), lambda i: (i, 0, 0)),
            pl.BlockSpec((1, _F0), lambda i: (0, 0)),
        ],
        out_specs=[
            pl.BlockSpec((1, _N, _F0), lambda i: (i, 0, 0)),
            pl.BlockSpec((1, 1, _N), lambda i: (i, 0, 0)),
        ],
        out_shape=[
            jax.ShapeDtypeStruct((_B * _T, _N, _F0), _f32),
            jax.ShapeDtypeStruct((_B * _T, 1, _N), _f32),
        ],
    )(x_tn, w)


def _adj_body(r_ref, c_ref, rk_ref, l_ref):
    eye = (_iota_r(_N) == _iota_c(_N)).astype(_f32)
    ic = _iota_c(_N)
    cnt = jnp.zeros((_N, _N), _f32)
    chunk = 2048
    for k in range(_E // chunk):
        rc = r_ref[k * chunk:(k + 1) * chunk, :]
        cc = c_ref[k * chunk:(k + 1) * chunk, :]
        ohr = (rc == ic).astype(_f32)
        ohc = (cc == ic).astype(_f32)
        cnt = cnt + _mm00(ohr, ohc, _DEF)
    deg = jnp.sum(cnt, axis=1, keepdims=True)
    dis = jnp.where(deg > 0, 1.0 / jnp.sqrt(deg), 0.0)
    dis_row = _mm00(dis, eye)
    l_ref[4] = -(dis * dis_row) * cnt
    for t in range(4):
        rk = rk_ref[t:t + 1, :]
        perm = (rk == _iota_r(_N)).astype(_f32)
        tmp = _mm(perm, cnt, _DEF)
        ct = _mm11(tmp, perm, _DEF)
        dt_col = _mm(perm, dis)
        dt_row = _mm00(dt_col, eye)
        l_ref[t] = -(dt_col * dt_row) * ct


def _adj_call(rows, cols, ranks7):
    return pl.pallas_call(
        _adj_body,
        grid=(1,),
        in_specs=[
            pl.BlockSpec((_E, 1), lambda i: (0, 0)),
            pl.BlockSpec((_E, 1), lambda i: (0, 0)),
            pl.BlockSpec((_B, _N), lambda i: (0, 0)),
        ],
        out_specs=pl.BlockSpec((5, _N, _N), lambda i: (0, 0, 0)),
        out_shape=jax.ShapeDtypeStruct((5, _N, _N), _f32),
    )(rows, cols, ranks7)


def _att_body(a_ref, u1_ref, u2_ref, u3_ref, be_ref, ve_ref,
              w1_ref, w2_ref, w3_ref, bs_ref, vs_ref, s_ref):
    A = a_ref[0]
    u1 = u1_ref[...]
    u3 = u3_ref[...]
    lhs1 = jnp.sum(A * u1[:, :, None], axis=1)
    lhs2 = _mm(lhs1, u2_ref[...])
    rhs = jnp.sum(A * u3[None, :, :], axis=2)
    e1 = _mm11(lhs2, rhs)
    esig = jax.nn.sigmoid(e1 + be_ref[...])
    eatt = _mm(ve_ref[...], esig)
    m = jnp.max(eatt, axis=0, keepdims=True)
    ex = jnp.exp(eatt - m)
    eatt = ex / jnp.sum(ex, axis=0, keepdims=True)
    c = _mm(eatt, w1_ref[...])
    l2a = jnp.sum(A * c[:, :, None], axis=0)
    l2b = _mm(l2a, w2_ref[...])
    r0 = jnp.sum(A * w3_ref[...][None, :, :], axis=2)
    r2 = _mm00(eatt, r0)
    s1 = jax.nn.sigmoid(_mm(l2b, r2) + bs_ref[...])
    s2 = _mm(vs_ref[...], s1)
    m2 = jnp.max(s2, axis=0, keepdims=True)
    ex2 = jnp.exp(s2 - m2)
    s_ref[0] = ex2 / jnp.sum(ex2, axis=0, keepdims=True)


def _att_call(A, p, F):
    return pl.pallas_call(
        _att_body,
        grid=(_B,),
        in_specs=[
            pl.BlockSpec((1, _T, _N, F), lambda b: (b, 0, 0, 0)),
            pl.BlockSpec((1, _N), lambda b: (0, 0)),
            pl.BlockSpec((F, _N), lambda b: (0, 0)),
            pl.BlockSpec((1, F), lambda b: (0, 0)),
            pl.BlockSpec((_T, _T), lambda b: (0, 0)),
            pl.BlockSpec((_T, _T), lambda b: (0, 0)),
            pl.BlockSpec((_T, 1), lambda b: (0, 0)),
            pl.BlockSpec((F, _T), lambda b: (0, 0)),
            pl.BlockSpec((1, F), lambda b: (0, 0)),
            pl.BlockSpec((_N, _N), lambda b: (0, 0)),
            pl.BlockSpec((_N, _N), lambda b: (0, 0)),
        ],
        out_specs=pl.BlockSpec((1, _N, _N), lambda b: (b, 0, 0)),
        out_shape=jax.ShapeDtypeStruct((_B, _N, _N), _f32),
    )(A, p['U1'].reshape(1, _N), p['U2'], p['U3'].reshape(1, F),
      p['be'][0], p['Ve'], p['W1'].reshape(_T, 1), p['W2'],
      p['W3'].reshape(1, F), p['bs'][0], p['Vs'])


def _cheb_body(a_ref, s_ref, l_ref, th_ref, cb_ref, h_ref):
    att = s_ref[0]
    lt = l_ref[0]
    xt = a_ref[0, 0]
    eye = (_iota_r(_N) == _iota_c(_N)).astype(_f32)
    diag = jnp.sum(att * eye, axis=1, keepdims=True)
    t0 = xt * diag
    out = _mm(t0, th_ref[0]) + cb_ref[...]
    t1 = _mm00(lt * att, t0)
    out = out + _mm(t1, th_ref[1])
    t2 = 2.0 * _mm00(lt, t1) - t0
    out = out + _mm(t2, th_ref[2])
    h_ref[0, 0] = jnp.maximum(out, 0.0)


def _cheb_call(A, S, L5, theta, cb, F):
    return pl.pallas_call(
        _cheb_body,
        grid=(_B, _T),
        in_specs=[
            pl.BlockSpec((1, 1, _N, F), lambda b, t: (b, t, 0, 0)),
            pl.BlockSpec((1, _N, _N), lambda b, t: (b, 0, 0)),
            pl.BlockSpec((1, _N, _N), lambda b, t: (jnp.minimum(t, 4), 0, 0)),
            pl.BlockSpec((_K, F, _C), lambda b, t: (0, 0, 0)),
            pl.BlockSpec((1, _C), lambda b, t: (0, 0)),
        ],
        out_specs=pl.BlockSpec((1, 1, _N, _C), lambda b, t: (b, t, 0, 0)),
        out_shape=jax.ShapeDtypeStruct((_B, _T, _N, _C), _f32),
    )(A, S, L5, theta, cb.reshape(1, _C))


def _conv_body(h_ref, a_ref, wt_ref, bt_ref, wr_ref, br_ref,
               g_ref, be_ref, o_ref):
    bt = bt_ref[...]
    br = br_ref[...]
    g = g_ref[...]
    be = be_ref[...]
    for t in range(_T):
        acc = jnp.zeros((_N, _C), _f32)
        for dt in range(3):
            src = t + dt - 1
            if 0 <= src < _T:
                acc = acc + _mm(h_ref[0, src], wt_ref[dt])
        xr = _mm(a_ref[0, t], wr_ref[...])
        z = jnp.maximum(acc + bt + xr + br, 0.0)
        mu = jnp.mean(z, axis=1, keepdims=True)
        var = jnp.mean((z - mu) * (z - mu), axis=1, keepdims=True)
        o_ref[0, t] = (z - mu) / jnp.sqrt(var + 1e-5) * g + be


def _conv_call(H, A, p, F):
    wt_r = jnp.transpose(p['Wt'][:, :, 0, :], (2, 1, 0))
    wr_r = jnp.transpose(p['Wr'][:, :, 0, 0])
    return pl.pallas_call(
        _conv_body,
        grid=(_B,),
        in_specs=[
            pl.BlockSpec((1, _T, _N, _C), lambda b: (b, 0, 0, 0)),
            pl.BlockSpec((1, _T, _N, F), lambda b: (b, 0, 0, 0)),
            pl.BlockSpec((3, _C, _C), lambda b: (0, 0, 0)),
            pl.BlockSpec((1, _C), lambda b: (0, 0)),
            pl.BlockSpec((F, _C), lambda b: (0, 0)),
            pl.BlockSpec((1, _C), lambda b: (0, 0)),
            pl.BlockSpec((1, _C), lambda b: (0, 0)),
            pl.BlockSpec((1, _C), lambda b: (0, 0)),
        ],
        out_specs=pl.BlockSpec((1, _T, _N, _C), lambda b: (b, 0, 0, 0)),
        out_shape=jax.ShapeDtypeStruct((_B, _T, _N, _C), _f32),
    )(H, A, wt_r, p['bt'].reshape(1, _C), wr_r, p['br'].reshape(1, _C),
      p['gamma'].reshape(1, _C), p['beta'].reshape(1, _C))


def _head_body(a_ref, imu_ref, w1a_ref, w1b_ref, b1_ref, w2_ref, b2_ref,
               w3_ref, b3_ref, o_ref):
    xm = jnp.mean(a_ref[...], axis=2)
    h = _mm(xm, w1a_ref[...]) + _mm(imu_ref[...], w1b_ref[...]) + b1_ref[...]
    h = jnp.maximum(h, 0.0)
    h = jnp.maximum(_mm(h, w2_ref[...]) + b2_ref[...], 0.0)
    o_ref[...] = _mm(h, w3_ref[...]) + b3_ref[...]


def _head_call(a0, imu_flat, m):
    w1a = jnp.transpose(m['W1'][:, :_N])
    w1b = jnp.transpose(m['W1'][:, _N:])
    return pl.pallas_call(
        _head_body,
        grid=(1,),
        in_specs=[
            pl.BlockSpec((_B, _N, _C), lambda i: (0, 0, 0)),
            pl.BlockSpec((_B, 48), lambda i: (0, 0)),
            pl.BlockSpec((_N, 256), lambda i: (0, 0)),
            pl.BlockSpec((48, 256), lambda i: (0, 0)),
            pl.BlockSpec((1, 256), lambda i: (0, 0)),
            pl.BlockSpec((256, 128), lambda i: (0, 0)),
            pl.BlockSpec((1, 128), lambda i: (0, 0)),
            pl.BlockSpec((128, 6), lambda i: (0, 0)),
            pl.BlockSpec((1, 6), lambda i: (0, 0)),
        ],
        out_specs=pl.BlockSpec((_B, 6), lambda i: (0, 0)),
        out_shape=jax.ShapeDtypeStruct((_B, 6), _f32),
    )(a0, imu_flat, w1a, w1b, m['b1'].reshape(1, 256),
      jnp.transpose(m['W2']), m['b2'].reshape(1, 128),
      jnp.transpose(m['W3']), m['b3'].reshape(1, 6))


def kernel(x, edge_index, imu_data, params):
    x_tn = jnp.transpose(x, (0, 3, 1, 2)).reshape(_B * _T, _N, _F0)
    pooled, ranks = _pool_call(x_tn, params['pool_w'].reshape(1, _F0))
    A = pooled.reshape(_B, _T, _N, _F0)
    ranks7 = ranks.reshape(_B, _T, _N)[:, _T - 1, :]
    ei = edge_index.astype(jnp.int32)
    rows = ei[0].reshape(_E, 1).astype(_f32)
    cols = ei[1].reshape(_E, 1).astype(_f32)
    L5 = _adj_call(rows, cols, ranks7)
    F = _F0
    for bp in params['blocks']:
        S = _att_call(A, bp, F)
        H = _cheb_call(A, S, L5, bp['theta'], bp['cb'], F)
        A = _conv_call(H, A, bp, F)
        F = _C
    a0 = A[:, 0]
    imu_flat = imu_data.reshape(_B, 6 * _T)
    return _head_call(a0, imu_flat, params['mlp'])


# speedup vs baseline: 50.1553x; 1.0233x over previous
import jax
import jax.numpy as jnp
from jax import lax
from jax.experimental import pallas as pl
from jax.experimental.pallas import tpu as pltpu

_N = 512
_F0 = 8
_T = 8
_B = 4
_E = 8192
_C = 64
_K = 3
_f32 = jnp.float32
_HI = lax.Precision.HIGHEST
_DEF = lax.Precision.DEFAULT

_NBP = 18


def _mm(a, b, prec=_HI):
    return lax.dot_general(a, b, (((a.ndim - 1,), (0,)), ((), ())),
                           precision=prec, preferred_element_type=_f32)


def _mm00(a, b, prec=_HI):
    return lax.dot_general(a, b, (((0,), (0,)), ((), ())),
                           precision=prec, preferred_element_type=_f32)


def _mm11(a, b, prec=_HI):
    return lax.dot_general(a, b, (((1,), (1,)), ((), ())),
                           precision=prec, preferred_element_type=_f32)


def _iota_r(n):
    return lax.broadcasted_iota(jnp.int32, (n, 1), 0).astype(_f32)


def _iota_c(n):
    return lax.broadcasted_iota(jnp.int32, (1, n), 1).astype(_f32)


def _pool_body(x_ref, w_ref, out_ref, rank_ref):
    xb = x_ref[0]
    w = w_ref[...]
    nrm = jnp.sqrt(jnp.sum(w * w))
    s = jnp.tanh(jnp.sum(xb * w, axis=1, keepdims=True) / nrm)
    eye = (_iota_r(_N) == _iota_c(_N)).astype(_f32)
    s_row = _mm00(s, eye)
    gt = (s > s_row).astype(_f32)
    tie = ((s == s_row) & (_iota_r(_N) < _iota_c(_N))).astype(_f32)
    rank = jnp.sum(gt + tie, axis=0, keepdims=True)
    perm = (rank == _iota_r(_N)).astype(_f32)
    pooled = _mm(perm, xb * s)
    out_ref[0] = jnp.concatenate(
        [pooled, jnp.zeros((_N, _C - _F0), _f32)], axis=1)
    rank_ref[0] = rank


def _pool_call(x_tn, w):
    return pl.pallas_call(
        _pool_body,
        grid=(_B * _T,),
        in_specs=[
            pl.BlockSpec((1, _N, _F0), lambda i: (i, 0, 0)),
            pl.BlockSpec((1, _F0), lambda i: (0, 0)),
        ],
        out_specs=[
            pl.BlockSpec((1, _N, _C), lambda i: (i, 0, 0)),
            pl.BlockSpec((1, 1, _N), lambda i: (i, 0, 0)),
        ],
        out_shape=[
            jax.ShapeDtypeStruct((_B * _T, _N, _C), _f32),
            jax.ShapeDtypeStruct((_B * _T, 1, _N), _f32),
        ],
    )(x_tn, w)


def _adj_body(r_ref, c_ref, rk_ref, l_ref):
    eye = (_iota_r(_N) == _iota_c(_N)).astype(_f32)
    ic = _iota_c(_N)
    cnt = jnp.zeros((_N, _N), _f32)
    chunk = 2048
    for k in range(_E // chunk):
        rc = r_ref[k * chunk:(k + 1) * chunk, :]
        cc = c_ref[k * chunk:(k + 1) * chunk, :]
        ohr = (rc == ic).astype(_f32)
        ohc = (cc == ic).astype(_f32)
        cnt = cnt + _mm00(ohr, ohc, _DEF)
    deg = jnp.sum(cnt, axis=1, keepdims=True)
    dis = jnp.where(deg > 0, 1.0 / jnp.sqrt(deg), 0.0)
    dis_row = _mm00(dis, eye)
    l_ref[4] = -(dis * dis_row) * cnt
    for t in range(4):
        rk = rk_ref[t:t + 1, :]
        perm = (rk == _iota_r(_N)).astype(_f32)
        tmp = _mm(perm, cnt, _DEF)
        ct = _mm11(tmp, perm, _DEF)
        dt_col = _mm(perm, dis)
        dt_row = _mm00(dt_col, eye)
        l_ref[t] = -(dt_col * dt_row) * ct


def _adj_call(rows, cols, ranks7):
    return pl.pallas_call(
        _adj_body,
        grid=(1,),
        in_specs=[
            pl.BlockSpec((_E, 1), lambda i: (0, 0)),
            pl.BlockSpec((_E, 1), lambda i: (0, 0)),
            pl.BlockSpec((_B, _N), lambda i: (0, 0)),
        ],
        out_specs=pl.BlockSpec((5, _N, _N), lambda i: (0, 0, 0)),
        out_shape=jax.ShapeDtypeStruct((5, _N, _N), _f32),
    )(rows, cols, ranks7)


def _blocks_body(*refs):
    a0_ref, l_ref, imu_ref = refs[0:3]
    bps = refs[3:3 + 3 * _NBP]
    mlp = refs[3 + 3 * _NBP:3 + 3 * _NBP + 7]
    o_ref = refs[3 + 3 * _NBP + 7]
    a_ref, h_ref = refs[3 + 3 * _NBP + 8:]

    eye = (_iota_r(_N) == _iota_c(_N)).astype(_f32)

    for blk in range(3):
        (u1_ref, u2_ref, u3_ref, be_ref, ve_ref, w1_ref, w2_ref, w3_ref,
         bs_ref, vs_ref, th_ref, cb_ref, wt_ref, bt_ref, wr_ref, br_ref,
         g_ref, be2_ref) = bps[_NBP * blk:_NBP * (blk + 1)]
        src = a0_ref if blk == 0 else a_ref
        cb = cb_ref[...]
        btv = bt_ref[...]
        brv = br_ref[...]
        g = g_ref[...]
        be2 = be2_ref[...]
        u1 = u1_ref[...]
        u3 = u3_ref[...]
        w3 = w3_ref[...]

        def b_body(b, _):
            Ab = src[b]
            lhs1 = jnp.sum(Ab * u1[:, :, None], axis=1)
            lhs2 = _mm(lhs1, u2_ref[...])
            rhs = jnp.sum(Ab * u3[None, :, :], axis=2)
            e1 = _mm11(lhs2, rhs)
            esig = jax.nn.sigmoid(e1 + be_ref[...])
            eatt = _mm(ve_ref[...], esig)
            mx = jnp.max(eatt, axis=0, keepdims=True)
            ex = jnp.exp(eatt - mx)
            eatt = ex / jnp.sum(ex, axis=0, keepdims=True)
            c = _mm(eatt, w1_ref[...])
            l2a = jnp.sum(Ab * c[:, :, None], axis=0)
            l2b = _mm(l2a, w2_ref[...])
            r0 = jnp.sum(Ab * w3[None, :, :], axis=2)
            r2 = _mm00(eatt, r0)
            s1 = jax.nn.sigmoid(_mm(l2b, r2) + bs_ref[...])
            s2 = _mm(vs_ref[...], s1)
            mx2 = jnp.max(s2, axis=0, keepdims=True)
            ex2 = jnp.exp(s2 - mx2)
            S = ex2 / jnp.sum(ex2, axis=0, keepdims=True)
            diag = jnp.sum(S * eye, axis=1, keepdims=True)

            def t_cheb(t, _c):
                lt = l_ref[jnp.minimum(t, 4)]
                xt = src[b, t]
                t0 = xt * diag
                out = _mm(t0, th_ref[0]) + cb
                t1 = _mm00(lt * S, t0)
                out = out + _mm(t1, th_ref[1])
                t2 = 2.0 * _mm00(lt, t1) - t0
                out = out + _mm(t2, th_ref[2])
                h_ref[t] = jnp.maximum(out, 0.0)
                return 0
            lax.fori_loop(0, _T, t_cheb, 0)

            def t_conv(t, _c):
                acc = _mm(src[b, t], wr_ref[...])
                for dt in range(3):
                    stp = t + dt - 1
                    valid = jnp.logical_and(stp >= 0, stp < _T)
                    hs = h_ref[jnp.clip(stp, 0, _T - 1)]
                    acc = acc + jnp.where(valid, 1.0, 0.0) * _mm(hs, wt_ref[dt])
                z = jnp.maximum(acc + btv + brv, 0.0)
                mu = jnp.mean(z, axis=1, keepdims=True)
                var = jnp.mean((z - mu) * (z - mu), axis=1, keepdims=True)
                a_ref[b, t] = (z - mu) / jnp.sqrt(var + 1e-5) * g + be2
                return 0
            lax.fori_loop(0, _T, t_conv, 0)
            return 0

        lax.fori_loop(0, _B, b_body, 0)

    (w1a_ref, w1b_ref, b1_ref, w2m_ref, b2_ref, w3m_ref, b3_ref) = mlp
    xrows = []
    for b in range(_B):
        m = jnp.mean(a_ref[b, 0], axis=1, keepdims=True)
        xrows.append(_mm00(m, eye))
    xm = jnp.concatenate(xrows, axis=0)
    h = _mm(xm, w1a_ref[...]) + _mm(imu_ref[...], w1b_ref[...]) + b1_ref[...]
    h = jnp.maximum(h, 0.0)
    h = jnp.maximum(_mm(h, w2m_ref[...]) + b2_ref[...], 0.0)
    o_ref[...] = _mm(h, w3m_ref[...]) + b3_ref[...]


def _blocks_call(A0, L5, imu_flat, flat):
    n_in = 3 + len(flat)
    return pl.pallas_call(
        _blocks_body,
        out_shape=jax.ShapeDtypeStruct((_B, 6), _f32),
        scratch_shapes=[
            pltpu.VMEM((_B, _T, _N, _C), _f32),
            pltpu.VMEM((_T, _N, _C), _f32),
        ],
    )(A0, L5, imu_flat, *flat)


def _padF(a, axis):
    pad = [(0, 0)] * a.ndim
    pad[axis] = (0, _C - _F0)
    return jnp.pad(a, pad)


def kernel(x, edge_index, imu_data, params):
    x_tn = jnp.transpose(x, (0, 3, 1, 2)).reshape(_B * _T, _N, _F0)
    pooled, ranks = _pool_call(x_tn, params['pool_w'].reshape(1, _F0))
    A0 = pooled.reshape(_B, _T, _N, _C)
    ranks7 = ranks.reshape(_B, _T, _N)[:, _T - 1, :]
    ei = edge_index.astype(jnp.int32)
    rows = ei[0].reshape(_E, 1).astype(_f32)
    cols = ei[1].reshape(_E, 1).astype(_f32)
    L5 = _adj_call(rows, cols, ranks7)

    flat = []
    first = True
    for bp in params['blocks']:
        if first:
            u2 = _padF(bp['U2'], 0)
            u3 = _padF(bp['U3'].reshape(1, _F0), 1)
            w2 = _padF(bp['W2'], 0)
            w3 = _padF(bp['W3'].reshape(1, _F0), 1)
            th = _padF(bp['theta'], 1)
            wr = _padF(jnp.transpose(bp['Wr'][:, :, 0, 0]), 0)
            first = False
        else:
            u2, w2, th = bp['U2'], bp['W2'], bp['theta']
            u3 = bp['U3'].reshape(1, _C)
            w3 = bp['W3'].reshape(1, _C)
            wr = jnp.transpose(bp['Wr'][:, :, 0, 0])
        flat += [
            bp['U1'].reshape(1, _N), u2, u3,
            bp['be'][0], bp['Ve'], bp['W1'].reshape(_T, 1), w2, w3,
            bp['bs'][0], bp['Vs'], th, bp['cb'].reshape(1, _C),
            jnp.transpose(bp['Wt'][:, :, 0, :], (2, 1, 0)),
            bp['bt'].reshape(1, _C), wr, bp['br'].reshape(1, _C),
            bp['gamma'].reshape(1, _C), bp['beta'].reshape(1, _C),
        ]
    m = params['mlp']
    flat += [
        jnp.transpose(m['W1'][:, :_N]), jnp.transpose(m['W1'][:, _N:]),
        m['b1'].reshape(1, 256), jnp.transpose(m['W2']),
        m['b2'].reshape(1, 128), jnp.transpose(m['W3']),
        m['b3'].reshape(1, 6),
    ]
    imu_flat = imu_data.reshape(_B, 6 * _T)
    return _blocks_call(A0, L5, imu_flat, flat)


# speedup vs baseline: 114.4945x; 2.2828x over previous
import jax
import jax.numpy as jnp
from jax import lax
from jax.experimental import pallas as pl
from jax.experimental.pallas import tpu as pltpu

_N = 512
_F0 = 8
_T = 8
_B = 4
_E = 8192
_C = 64
_K = 3
_f32 = jnp.float32
_HI = lax.Precision.HIGHEST
_DEF = lax.Precision.DEFAULT

_NBP = 18


def _mm(a, b, prec=_DEF):
    return lax.dot_general(a, b, (((a.ndim - 1,), (0,)), ((), ())),
                           precision=prec, preferred_element_type=_f32)


def _mm00(a, b, prec=_DEF):
    return lax.dot_general(a, b, (((0,), (0,)), ((), ())),
                           precision=prec, preferred_element_type=_f32)


def _mm11(a, b, prec=_DEF):
    return lax.dot_general(a, b, (((1,), (1,)), ((), ())),
                           precision=prec, preferred_element_type=_f32)


def _iota_r(n):
    return lax.broadcasted_iota(jnp.int32, (n, 1), 0).astype(_f32)


def _iota_c(n):
    return lax.broadcasted_iota(jnp.int32, (1, n), 1).astype(_f32)


def _pool_body(x_ref, w_ref, out_ref, rank_ref):
    xb = x_ref[0]
    w = w_ref[...]
    nrm = jnp.sqrt(jnp.sum(w * w))
    s = jnp.tanh(jnp.sum(xb * w, axis=1, keepdims=True) / nrm)
    eye = (_iota_r(_N) == _iota_c(_N)).astype(_f32)
    s_row = _mm00(s, eye, _HI)
    gt = (s > s_row).astype(_f32)
    tie = ((s == s_row) & (_iota_r(_N) < _iota_c(_N))).astype(_f32)
    rank = jnp.sum(gt + tie, axis=0, keepdims=True)
    perm = (rank == _iota_r(_N)).astype(_f32)
    pooled = _mm(perm, xb * s, _HI)
    out_ref[0] = jnp.concatenate(
        [pooled, jnp.zeros((_N, _C - _F0), _f32)], axis=1)
    rank_ref[0] = rank


def _pool_call(x_tn, w):
    return pl.pallas_call(
        _pool_body,
        grid=(_B * _T,),
        in_specs=[
            pl.BlockSpec((1, _N, _F0), lambda i: (i, 0, 0)),
            pl.BlockSpec((1, _F0), lambda i: (0, 0)),
        ],
        out_specs=[
            pl.BlockSpec((1, _N, _C), lambda i: (i, 0, 0)),
            pl.BlockSpec((1, 1, _N), lambda i: (i, 0, 0)),
        ],
        out_shape=[
            jax.ShapeDtypeStruct((_B * _T, _N, _C), _f32),
            jax.ShapeDtypeStruct((_B * _T, 1, _N), _f32),
        ],
    )(x_tn, w)


def _adj_body(r_ref, c_ref, rk_ref, l_ref):
    eye = (_iota_r(_N) == _iota_c(_N)).astype(_f32)
    ic = _iota_c(_N)
    cnt = jnp.zeros((_N, _N), _f32)
    chunk = 2048
    for k in range(_E // chunk):
        rc = r_ref[k * chunk:(k + 1) * chunk, :]
        cc = c_ref[k * chunk:(k + 1) * chunk, :]
        ohr = (rc == ic).astype(_f32)
        ohc = (cc == ic).astype(_f32)
        cnt = cnt + _mm00(ohr, ohc, _DEF)
    deg = jnp.sum(cnt, axis=1, keepdims=True)
    dis = jnp.where(deg > 0, 1.0 / jnp.sqrt(deg), 0.0)
    dis_row = _mm00(dis, eye, _HI)
    l_ref[4] = -(dis * dis_row) * cnt
    for t in range(4):
        rk = rk_ref[t:t + 1, :]
        perm = (rk == _iota_r(_N)).astype(_f32)
        tmp = _mm(perm, cnt, _DEF)
        ct = _mm11(tmp, perm, _DEF)
        dt_col = _mm(perm, dis, _HI)
        dt_row = _mm00(dt_col, eye, _HI)
        l_ref[t] = -(dt_col * dt_row) * ct


def _adj_call(rows, cols, ranks7):
    return pl.pallas_call(
        _adj_body,
        grid=(1,),
        in_specs=[
            pl.BlockSpec((_E, 1), lambda i: (0, 0)),
            pl.BlockSpec((_E, 1), lambda i: (0, 0)),
            pl.BlockSpec((_B, _N), lambda i: (0, 0)),
        ],
        out_specs=pl.BlockSpec((5, _N, _N), lambda i: (0, 0, 0)),
        out_shape=jax.ShapeDtypeStruct((5, _N, _N), _f32),
    )(rows, cols, ranks7)


def _blocks_body(*refs):
    a0_ref, l_ref, imu_ref = refs[0:3]
    bps = refs[3:3 + 3 * _NBP]
    mlp = refs[3 + 3 * _NBP:3 + 3 * _NBP + 7]
    o_ref = refs[3 + 3 * _NBP + 7]
    a_ref, h_ref = refs[3 + 3 * _NBP + 8:]

    eye = (_iota_r(_N) == _iota_c(_N)).astype(_f32)

    for blk in range(3):
        (u1_ref, u2_ref, u3_ref, be_ref, ve_ref, w1_ref, w2_ref, w3_ref,
         bs_ref, vs_ref, th_ref, cb_ref, wt_ref, bt_ref, wr_ref, br_ref,
         g_ref, be2_ref) = bps[_NBP * blk:_NBP * (blk + 1)]
        src = a0_ref if blk == 0 else a_ref
        cb = cb_ref[...]
        btv = bt_ref[...]
        brv = br_ref[...]
        g = g_ref[...]
        be2 = be2_ref[...]
        u1 = u1_ref[...]
        u3 = u3_ref[...]
        w3 = w3_ref[...]

        def b_body(b, _):
            Ab = src[b]
            lhs1 = jnp.sum(Ab * u1[:, :, None], axis=1)
            lhs2 = _mm(lhs1, u2_ref[...])
            rhs = jnp.sum(Ab * u3[None, :, :], axis=2)
            e1 = _mm11(lhs2, rhs)
            esig = jax.nn.sigmoid(e1 + be_ref[...])
            eatt = _mm(ve_ref[...], esig)
            mx = jnp.max(eatt, axis=0, keepdims=True)
            ex = jnp.exp(eatt - mx)
            eatt = ex / jnp.sum(ex, axis=0, keepdims=True)
            c = _mm(eatt, w1_ref[...])
            l2a = jnp.sum(Ab * c[:, :, None], axis=0)
            l2b = _mm(l2a, w2_ref[...])
            r0 = jnp.sum(Ab * w3[None, :, :], axis=2)
            r2 = _mm00(eatt, r0)
            s1 = jax.nn.sigmoid(_mm(l2b, r2) + bs_ref[...])
            s2 = _mm(vs_ref[...], s1)
            mx2 = jnp.max(s2, axis=0, keepdims=True)
            ex2 = jnp.exp(s2 - mx2)
            S = ex2 / jnp.sum(ex2, axis=0, keepdims=True)
            diag = jnp.sum(S * eye, axis=1, keepdims=True)

            def t_cheb(t, _c):
                lt = l_ref[jnp.minimum(t, 4)]
                xt = src[b, t]
                t0 = xt * diag
                out = _mm(t0, th_ref[0]) + cb
                t1 = _mm00(lt * S, t0)
                out = out + _mm(t1, th_ref[1])
                t2 = 2.0 * _mm00(lt, t1) - t0
                out = out + _mm(t2, th_ref[2])
                h_ref[t] = jnp.maximum(out, 0.0)
                return 0
            lax.fori_loop(0, _T, t_cheb, 0)

            def t_conv(t, _c):
                acc = _mm(src[b, t], wr_ref[...])
                for dt in range(3):
                    stp = t + dt - 1
                    valid = jnp.logical_and(stp >= 0, stp < _T)
                    hs = h_ref[jnp.clip(stp, 0, _T - 1)]
                    acc = acc + jnp.where(valid, 1.0, 0.0) * _mm(hs, wt_ref[dt])
                z = jnp.maximum(acc + btv + brv, 0.0)
                mu = jnp.mean(z, axis=1, keepdims=True)
                var = jnp.mean((z - mu) * (z - mu), axis=1, keepdims=True)
                a_ref[b, t] = (z - mu) / jnp.sqrt(var + 1e-5) * g + be2
                return 0
            lax.fori_loop(0, _T, t_conv, 0)
            return 0

        lax.fori_loop(0, _B, b_body, 0)

    (w1a_ref, w1b_ref, b1_ref, w2m_ref, b2_ref, w3m_ref, b3_ref) = mlp
    xrows = []
    for b in range(_B):
        m = jnp.mean(a_ref[b, 0], axis=1, keepdims=True)
        xrows.append(_mm00(m, eye, _HI))
    xm = jnp.concatenate(xrows, axis=0)
    h = _mm(xm, w1a_ref[...]) + _mm(imu_ref[...], w1b_ref[...]) + b1_ref[...]
    h = jnp.maximum(h, 0.0)
    h = jnp.maximum(_mm(h, w2m_ref[...]) + b2_ref[...], 0.0)
    o_ref[...] = _mm(h, w3m_ref[...]) + b3_ref[...]


def _blocks_call(A0, L5, imu_flat, flat):
    n_in = 3 + len(flat)
    return pl.pallas_call(
        _blocks_body,
        out_shape=jax.ShapeDtypeStruct((_B, 6), _f32),
        scratch_shapes=[
            pltpu.VMEM((_B, _T, _N, _C), _f32),
            pltpu.VMEM((_T, _N, _C), _f32),
        ],
    )(A0, L5, imu_flat, *flat)


def _padF(a, axis):
    pad = [(0, 0)] * a.ndim
    pad[axis] = (0, _C - _F0)
    return jnp.pad(a, pad)


def kernel(x, edge_index, imu_data, params):
    x_tn = jnp.transpose(x, (0, 3, 1, 2)).reshape(_B * _T, _N, _F0)
    pooled, ranks = _pool_call(x_tn, params['pool_w'].reshape(1, _F0))
    A0 = pooled.reshape(_B, _T, _N, _C)
    ranks7 = ranks.reshape(_B, _T, _N)[:, _T - 1, :]
    ei = edge_index.astype(jnp.int32)
    rows = ei[0].reshape(_E, 1).astype(_f32)
    cols = ei[1].reshape(_E, 1).astype(_f32)
    L5 = _adj_call(rows, cols, ranks7)

    flat = []
    first = True
    for bp in params['blocks']:
        if first:
            u2 = _padF(bp['U2'], 0)
            u3 = _padF(bp['U3'].reshape(1, _F0), 1)
            w2 = _padF(bp['W2'], 0)
            w3 = _padF(bp['W3'].reshape(1, _F0), 1)
            th = _padF(bp['theta'], 1)
            wr = _padF(jnp.transpose(bp['Wr'][:, :, 0, 0]), 0)
            first = False
        else:
            u2, w2, th = bp['U2'], bp['W2'], bp['theta']
            u3 = bp['U3'].reshape(1, _C)
            w3 = bp['W3'].reshape(1, _C)
            wr = jnp.transpose(bp['Wr'][:, :, 0, 0])
        flat += [
            bp['U1'].reshape(1, _N), u2, u3,
            bp['be'][0], bp['Ve'], bp['W1'].reshape(_T, 1), w2, w3,
            bp['bs'][0], bp['Vs'], th, bp['cb'].reshape(1, _C),
            jnp.transpose(bp['Wt'][:, :, 0, :], (2, 1, 0)),
            bp['bt'].reshape(1, _C), wr, bp['br'].reshape(1, _C),
            bp['gamma'].reshape(1, _C), bp['beta'].reshape(1, _C),
        ]
    m = params['mlp']
    flat += [
        jnp.transpose(m['W1'][:, :_N]), jnp.transpose(m['W1'][:, _N:]),
        m['b1'].reshape(1, 256), jnp.transpose(m['W2']),
        m['b2'].reshape(1, 128), jnp.transpose(m['W3']),
        m['b3'].reshape(1, 6),
    ]
    imu_flat = imu_data.reshape(_B, 6 * _T)
    return _blocks_call(A0, L5, imu_flat, flat)


# speedup vs baseline: 135.8913x; 1.1869x over previous
import jax
import jax.numpy as jnp
from jax import lax
from jax.experimental import pallas as pl
from jax.experimental.pallas import tpu as pltpu

_N = 512
_F0 = 8
_T = 8
_B = 4
_E = 8192
_C = 64
_K = 3
_f32 = jnp.float32
_HI = lax.Precision.HIGHEST
_DEF = lax.Precision.DEFAULT

_NBP = 18


def _mm(a, b, prec=_DEF):
    return lax.dot_general(a, b, (((a.ndim - 1,), (0,)), ((), ())),
                           precision=prec, preferred_element_type=_f32)


def _mm00(a, b, prec=_DEF):
    return lax.dot_general(a, b, (((0,), (0,)), ((), ())),
                           precision=prec, preferred_element_type=_f32)


def _mm11(a, b, prec=_DEF):
    return lax.dot_general(a, b, (((1,), (1,)), ((), ())),
                           precision=prec, preferred_element_type=_f32)


def _iota_r(n):
    return lax.broadcasted_iota(jnp.int32, (n, 1), 0).astype(_f32)


def _iota_c(n):
    return lax.broadcasted_iota(jnp.int32, (1, n), 1).astype(_f32)


def _pool_body(x_ref, w_ref, out_ref, rank_ref):
    xb = x_ref[0]
    w = w_ref[...]
    nrm = jnp.sqrt(jnp.sum(w * w))
    s = jnp.tanh(jnp.sum(xb * w, axis=1, keepdims=True) / nrm)
    eye = (_iota_r(_N) == _iota_c(_N)).astype(_f32)
    s_row = _mm00(s, eye)
    gt = (s > s_row).astype(_f32)
    tie = ((s == s_row) & (_iota_r(_N) < _iota_c(_N))).astype(_f32)
    rank = jnp.sum(gt + tie, axis=0, keepdims=True)
    perm = (rank == _iota_r(_N)).astype(_f32)
    pooled = _mm(perm, xb * s)
    out_ref[0] = jnp.concatenate(
        [pooled, jnp.zeros((_N, _C - _F0), _f32)], axis=1)
    rank_ref[0] = rank


def _pool_call(x_tn, w):
    return pl.pallas_call(
        _pool_body,
        grid=(_B * _T,),
        in_specs=[
            pl.BlockSpec((1, _N, _F0), lambda i: (i, 0, 0)),
            pl.BlockSpec((1, _F0), lambda i: (0, 0)),
        ],
        out_specs=[
            pl.BlockSpec((1, _N, _C), lambda i: (i, 0, 0)),
            pl.BlockSpec((1, 1, _N), lambda i: (i, 0, 0)),
        ],
        out_shape=[
            jax.ShapeDtypeStruct((_B * _T, _N, _C), _f32),
            jax.ShapeDtypeStruct((_B * _T, 1, _N), _f32),
        ],
    )(x_tn, w)


def _adj_body(r_ref, c_ref, rk_ref, l_ref):
    eye = (_iota_r(_N) == _iota_c(_N)).astype(_f32)
    ic = _iota_c(_N)
    cnt = jnp.zeros((_N, _N), _f32)
    chunk = 2048
    for k in range(_E // chunk):
        rc = r_ref[k * chunk:(k + 1) * chunk, :]
        cc = c_ref[k * chunk:(k + 1) * chunk, :]
        ohr = (rc == ic).astype(_f32)
        ohc = (cc == ic).astype(_f32)
        cnt = cnt + _mm00(ohr, ohc, _DEF)
    deg = jnp.sum(cnt, axis=1, keepdims=True)
    dis = jnp.where(deg > 0, 1.0 / jnp.sqrt(deg), 0.0)
    dis_row = _mm00(dis, eye)
    l_ref[4] = -(dis * dis_row) * cnt
    for t in range(4):
        rk = rk_ref[t:t + 1, :]
        perm = (rk == _iota_r(_N)).astype(_f32)
        tmp = _mm(perm, cnt, _DEF)
        ct = _mm11(tmp, perm, _DEF)
        dt_col = _mm(perm, dis)
        dt_row = _mm00(dt_col, eye)
        l_ref[t] = -(dt_col * dt_row) * ct


def _adj_call(rows, cols, ranks7):
    return pl.pallas_call(
        _adj_body,
        grid=(1,),
        in_specs=[
            pl.BlockSpec((_E, 1), lambda i: (0, 0)),
            pl.BlockSpec((_E, 1), lambda i: (0, 0)),
            pl.BlockSpec((_B, _N), lambda i: (0, 0)),
        ],
        out_specs=pl.BlockSpec((5, _N, _N), lambda i: (0, 0, 0)),
        out_shape=jax.ShapeDtypeStruct((5, _N, _N), _f32),
    )(rows, cols, ranks7)


def _blocks_body(*refs):
    a0_ref, l_ref, imu_ref = refs[0:3]
    bps = refs[3:3 + 3 * _NBP]
    mlp = refs[3 + 3 * _NBP:3 + 3 * _NBP + 7]
    o_ref = refs[3 + 3 * _NBP + 7]
    a_ref, h_ref = refs[3 + 3 * _NBP + 8:]

    eye = (_iota_r(_N) == _iota_c(_N)).astype(_f32)

    for blk in range(3):
        (u1_ref, u2_ref, u3_ref, be_ref, ve_ref, w1_ref, w2_ref, w3_ref,
         bs_ref, vs_ref, th_ref, cb_ref, wt_ref, bt_ref, wr_ref, br_ref,
         g_ref, be2_ref) = bps[_NBP * blk:_NBP * (blk + 1)]
        src = a0_ref if blk == 0 else a_ref
        cb = cb_ref[...]
        btv = bt_ref[...]
        brv = br_ref[...]
        g = g_ref[...]
        be2 = be2_ref[...]
        u1 = u1_ref[...]
        u3 = u3_ref[...]
        w3 = w3_ref[...]

        def b_body(b, _):
            Ab = src[b]
            lhs1 = jnp.sum(Ab * u1[:, :, None], axis=1)
            lhs2 = _mm(lhs1, u2_ref[...])
            rhs = jnp.sum(Ab * u3[None, :, :], axis=2)
            e1 = _mm11(lhs2, rhs)
            esig = jax.nn.sigmoid(e1 + be_ref[...])
            eatt = _mm(ve_ref[...], esig)
            mx = jnp.max(eatt, axis=0, keepdims=True)
            ex = jnp.exp(eatt - mx)
            eatt = ex / jnp.sum(ex, axis=0, keepdims=True)
            c = _mm(eatt, w1_ref[...])
            l2a = jnp.sum(Ab * c[:, :, None], axis=0)
            l2b = _mm(l2a, w2_ref[...])
            r0 = jnp.sum(Ab * w3[None, :, :], axis=2)
            r2 = _mm00(eatt, r0)
            s1 = jax.nn.sigmoid(_mm(l2b, r2) + bs_ref[...])
            s2 = _mm(vs_ref[...], s1)
            mx2 = jnp.max(s2, axis=0, keepdims=True)
            ex2 = jnp.exp(s2 - mx2)
            S = ex2 / jnp.sum(ex2, axis=0, keepdims=True)
            diag = jnp.sum(S * eye, axis=1, keepdims=True)

            def t_cheb(t, _c):
                lt = l_ref[jnp.minimum(t, 4)]
                xt = src[b, t]
                t0 = xt * diag
                out = _mm(t0, th_ref[0]) + cb
                t1 = _mm00(lt * S, t0)
                out = out + _mm(t1, th_ref[1])
                t2 = 2.0 * _mm00(lt, t1) - t0
                out = out + _mm(t2, th_ref[2])
                h_ref[t] = jnp.maximum(out, 0.0)
                return 0
            lax.fori_loop(0, _T, t_cheb, 0)

            def t_conv(t, _c):
                acc = _mm(src[b, t], wr_ref[...])
                for dt in range(3):
                    stp = t + dt - 1
                    valid = jnp.logical_and(stp >= 0, stp < _T)
                    hs = h_ref[jnp.clip(stp, 0, _T - 1)]
                    acc = acc + jnp.where(valid, 1.0, 0.0) * _mm(hs, wt_ref[dt])
                z = jnp.maximum(acc + btv + brv, 0.0)
                mu = jnp.mean(z, axis=1, keepdims=True)
                var = jnp.mean((z - mu) * (z - mu), axis=1, keepdims=True)
                a_ref[b, t] = (z - mu) / jnp.sqrt(var + 1e-5) * g + be2
                return 0
            lax.fori_loop(0, _T, t_conv, 0)
            return 0

        lax.fori_loop(0, _B, b_body, 0)

    (w1a_ref, w1b_ref, b1_ref, w2m_ref, b2_ref, w3m_ref, b3_ref) = mlp
    xrows = []
    for b in range(_B):
        m = jnp.mean(a_ref[b, 0], axis=1, keepdims=True)
        xrows.append(_mm00(m, eye))
    xm = jnp.concatenate(xrows, axis=0)
    h = _mm(xm, w1a_ref[...]) + _mm(imu_ref[...], w1b_ref[...]) + b1_ref[...]
    h = jnp.maximum(h, 0.0)
    h = jnp.maximum(_mm(h, w2m_ref[...]) + b2_ref[...], 0.0)
    o_ref[...] = _mm(h, w3m_ref[...]) + b3_ref[...]


def _blocks_call(A0, L5, imu_flat, flat):
    n_in = 3 + len(flat)
    return pl.pallas_call(
        _blocks_body,
        out_shape=jax.ShapeDtypeStruct((_B, 6), _f32),
        scratch_shapes=[
            pltpu.VMEM((_B, _T, _N, _C), _f32),
            pltpu.VMEM((_T, _N, _C), _f32),
        ],
    )(A0, L5, imu_flat, *flat)


def _padF(a, axis):
    pad = [(0, 0)] * a.ndim
    pad[axis] = (0, _C - _F0)
    return jnp.pad(a, pad)


def kernel(x, edge_index, imu_data, params):
    x_tn = jnp.transpose(x, (0, 3, 1, 2)).reshape(_B * _T, _N, _F0)
    pooled, ranks = _pool_call(x_tn, params['pool_w'].reshape(1, _F0))
    A0 = pooled.reshape(_B, _T, _N, _C)
    ranks7 = ranks.reshape(_B, _T, _N)[:, _T - 1, :]
    ei = edge_index.astype(jnp.int32)
    rows = ei[0].reshape(_E, 1).astype(_f32)
    cols = ei[1].reshape(_E, 1).astype(_f32)
    L5 = _adj_call(rows, cols, ranks7)

    flat = []
    first = True
    for bp in params['blocks']:
        if first:
            u2 = _padF(bp['U2'], 0)
            u3 = _padF(bp['U3'].reshape(1, _F0), 1)
            w2 = _padF(bp['W2'], 0)
            w3 = _padF(bp['W3'].reshape(1, _F0), 1)
            th = _padF(bp['theta'], 1)
            wr = _padF(jnp.transpose(bp['Wr'][:, :, 0, 0]), 0)
            first = False
        else:
            u2, w2, th = bp['U2'], bp['W2'], bp['theta']
            u3 = bp['U3'].reshape(1, _C)
            w3 = bp['W3'].reshape(1, _C)
            wr = jnp.transpose(bp['Wr'][:, :, 0, 0])
        flat += [
            bp['U1'].reshape(1, _N), u2, u3,
            bp['be'][0], bp['Ve'], bp['W1'].reshape(_T, 1), w2, w3,
            bp['bs'][0], bp['Vs'], th, bp['cb'].reshape(1, _C),
            jnp.transpose(bp['Wt'][:, :, 0, :], (2, 1, 0)),
            bp['bt'].reshape(1, _C), wr, bp['br'].reshape(1, _C),
            bp['gamma'].reshape(1, _C), bp['beta'].reshape(1, _C),
        ]
    m = params['mlp']
    flat += [
        jnp.transpose(m['W1'][:, :_N]), jnp.transpose(m['W1'][:, _N:]),
        m['b1'].reshape(1, 256), jnp.transpose(m['W2']),
        m['b2'].reshape(1, 128), jnp.transpose(m['W3']),
        m['b3'].reshape(1, 6),
    ]
    imu_flat = imu_data.reshape(_B, 6 * _T)
    return _blocks_call(A0, L5, imu_flat, flat)
